# baseline (speedup 1.0000x reference)
import functools

import jax
import jax.numpy as jnp
from jax import lax
from jax.experimental import pallas as pl
from jax.experimental.pallas import tpu as pltpu
from jax.experimental.pallas import tpu_sc as plsc

NC = 2
NS = 16
NW = NC * NS
CH = 128


def _dense_body(e, n, echunk, h_ref, ei_ref, wg_ref, wa_ref, ba_ref, wb_ref,
                bb_ref, wc_ref, bc_ref, g_ref, a_ref, c_ref, src_ref,
                dst_ref):
    h = h_ref[...]
    g_ref[...] = jnp.dot(h, wg_ref[...], preferred_element_type=jnp.float32)
    a = jnp.dot(h, wa_ref[...], preferred_element_type=jnp.float32) + ba_ref[...]
    a_ref[...] = jnp.maximum(a, 0.0)
    tb = jnp.dot(h, wb_ref[...], preferred_element_type=jnp.float32) + bb_ref[...]
    tc = jnp.dot(h, wc_ref[...], preferred_element_type=jnp.float32) + bc_ref[...]
    c_ref[...] = jnp.maximum(tb * tc, 0.0)
    i = pl.program_id(0)
    pos = i * echunk + jax.lax.broadcasted_iota(jnp.int32, (1, echunk), 1)
    valid = pos < e
    eb = ei_ref[...]
    src_ref[...] = jnp.where(valid, eb[0:1, :], 0).reshape(1, 1, echunk)
    dst_ref[...] = jnp.where(valid, eb[1:2, :], n).reshape(1, 1, echunk)


def _dense(h, ei, e_pad, wg, wa, ba, wb, bb, wc, bc, bn):
    n, fan = h.shape
    e = ei.shape[1]
    nout = wg.shape[1]
    grid = n // bn
    echunk = e_pad // grid
    full = lambda i: (0, 0)
    row = lambda i: (i, 0)
    return pl.pallas_call(
        functools.partial(_dense_body, e, n, echunk),
        grid=(grid,),
        in_specs=[
            pl.BlockSpec((bn, fan), row),
            pl.BlockSpec((2, echunk), lambda i: (0, i)),
            pl.BlockSpec((fan, nout), full),
            pl.BlockSpec((fan, nout), full),
            pl.BlockSpec((1, nout), full),
            pl.BlockSpec((fan, nout), full),
            pl.BlockSpec((1, nout), full),
            pl.BlockSpec((fan, nout), full),
            pl.BlockSpec((1, nout), full),
        ],
        out_specs=[
            pl.BlockSpec((bn, nout), row),
            pl.BlockSpec((bn, nout), row),
            pl.BlockSpec((bn, nout), row),
            pl.BlockSpec((1, 1, echunk), lambda i: (i, 0, 0)),
            pl.BlockSpec((1, 1, echunk), lambda i: (i, 0, 0)),
        ],
        out_shape=[jax.ShapeDtypeStruct((n, nout), jnp.float32)] * 3
        + [jax.ShapeDtypeStruct((grid, 1, echunk), jnp.int32)] * 2,
    )(h, ei, wg, wa, ba.reshape(1, -1), wb, bb.reshape(1, -1), wc,
      bc.reshape(1, -1))


def _fused_body(a_ref, agg_ref, c_ref, pbias_ref, wg_ref, wa_ref, ba_ref,
                wb_ref, bb_ref, wc_ref, bc_ref, g_ref, a_out_ref, c_out_ref):
    agg = agg_ref[...]
    b_ = jnp.maximum(agg[0] + agg[1] + pbias_ref[...], 0.0)
    h = jnp.concatenate([a_ref[...], b_, c_ref[...]], axis=1)
    g_ref[...] = jnp.dot(h, wg_ref[...], preferred_element_type=jnp.float32)
    a = jnp.dot(h, wa_ref[...], preferred_element_type=jnp.float32) + ba_ref[...]
    a_out_ref[...] = jnp.maximum(a, 0.0)
    tb = jnp.dot(h, wb_ref[...], preferred_element_type=jnp.float32) + bb_ref[...]
    tc = jnp.dot(h, wc_ref[...], preferred_element_type=jnp.float32) + bc_ref[...]
    c_out_ref[...] = jnp.maximum(tb * tc, 0.0)


def _fused_dense(a, agg, c, pbias, wg, wa, ba, wb, bb, wc, bc, bn):
    n, nout = a.shape
    fan = 3 * nout
    grid = n // bn
    full = lambda i: (0, 0)
    row = lambda i: (i, 0)
    return pl.pallas_call(
        _fused_body,
        grid=(grid,),
        in_specs=[
            pl.BlockSpec((bn, nout), row),
            pl.BlockSpec((2, bn, nout), lambda i: (0, i, 0)),
            pl.BlockSpec((bn, nout), row),
            pl.BlockSpec((1, nout), full),
            pl.BlockSpec((fan, nout), full),
            pl.BlockSpec((fan, nout), full),
            pl.BlockSpec((1, nout), full),
            pl.BlockSpec((fan, nout), full),
            pl.BlockSpec((1, nout), full),
            pl.BlockSpec((fan, nout), full),
            pl.BlockSpec((1, nout), full),
        ],
        out_specs=[
            pl.BlockSpec((bn, nout), row),
            pl.BlockSpec((bn, nout), row),
            pl.BlockSpec((bn, nout), row),
        ],
        out_shape=[jax.ShapeDtypeStruct((n, nout), jnp.float32)] * 3,
    )(a, agg, c, pbias.reshape(1, -1), wg, wa, ba.reshape(1, -1), wb,
      bb.reshape(1, -1), wc, bc.reshape(1, -1))


def _pool_body(a_ref, agg_ref, c_ref, bias_ref, batch_ref, w1_ref, b1_ref,
               w2_ref, b2_ref, out_ref, acc_ref):
    i = pl.program_id(0)
    agg = agg_ref[...]
    b_ = jnp.maximum(agg[0] + agg[1] + bias_ref[...], 0.0)
    h = jnp.concatenate([a_ref[...], b_, c_ref[...]], axis=1)
    gid = batch_ref[...]
    ng = acc_ref.shape[0]
    onehot = (gid == lax.broadcasted_iota(jnp.int32, (1, ng), 1)
              ).astype(jnp.float32)
    part = lax.dot_general(onehot, h, (((0,), (0,)), ((), ())),
                           preferred_element_type=jnp.float32)

    @pl.when(i == 0)
    def _():
        acc_ref[...] = jnp.zeros_like(acc_ref)

    acc_ref[...] += part

    @pl.when(i == pl.num_programs(0) - 1)
    def _():
        o = jnp.dot(acc_ref[...], w1_ref[...],
                    preferred_element_type=jnp.float32) + b1_ref[...]
        o = jnp.dot(o, w2_ref[...],
                    preferred_element_type=jnp.float32) + b2_ref[...]
        out_ref[...] = o


def _pool(a, agg, c, bias, batch2, w1, b1, w2, b2, ng, bn):
    n, nout = a.shape
    grid = n // bn
    nin = 3 * nout
    nh = w1.shape[1]
    return pl.pallas_call(
        _pool_body,
        grid=(grid,),
        in_specs=[
            pl.BlockSpec((bn, nout), lambda i: (i, 0)),
            pl.BlockSpec((2, bn, nout), lambda i: (0, i, 0)),
            pl.BlockSpec((bn, nout), lambda i: (i, 0)),
            pl.BlockSpec((1, nout), lambda i: (0, 0)),
            pl.BlockSpec((bn, 1), lambda i: (i, 0)),
            pl.BlockSpec((nin, nh), lambda i: (0, 0)),
            pl.BlockSpec((1, nh), lambda i: (0, 0)),
            pl.BlockSpec((nh, 1), lambda i: (0, 0)),
            pl.BlockSpec((1, 1), lambda i: (0, 0)),
        ],
        out_specs=pl.BlockSpec((ng, 1), lambda i: (0, 0)),
        out_shape=jax.ShapeDtypeStruct((ng, 1), jnp.float32),
        scratch_shapes=[pltpu.VMEM((ng, nin), jnp.float32)],
    )(a, agg, c, bias.reshape(1, -1), batch2, w1, b1.reshape(1, -1), w2,
      b2.reshape(1, -1))


NBUF = 8
KG = 4


def _make_scatter(n_pad, n_chunks, nout):
    rows_per = n_pad // NS
    ngroups = n_chunks // NBUF
    mesh = plsc.VectorSubcoreMesh(core_axis_name="c", subcore_axis_name="s")

    @functools.partial(
        pl.kernel, mesh=mesh,
        compiler_params=pltpu.CompilerParams(use_tc_tiling_on_sc=False),
        out_type=jax.ShapeDtypeStruct((NC, n_pad, nout), jnp.float32),
        scratch_types=[
            pltpu.VMEM((n_chunks, CH), jnp.int32),
            pltpu.VMEM((n_chunks, CH), jnp.int32),
            pltpu.VMEM((2, KG, CH, nout), jnp.float32),
            pltpu.VMEM((rows_per, nout), jnp.float32),
            pltpu.VMEM_SHARED((n_pad, nout), jnp.float32),
            pltpu.VMEM_SHARED((n_pad, nout), jnp.float32),
            pltpu.SemaphoreType.DMA,
            pltpu.SemaphoreType.DMA,
            pltpu.SemaphoreType.DMA,
            pltpu.SemaphoreType.DMA,
        ],
    )
    def scatter(g_hbm, src_hbm, dst_hbm, out_hbm,
                src_v, dst_v, gbuf, zbuf, acc, gsh, gsem_a, gsem_b, ssem_a,
                ssem_b):
        c = lax.axis_index("c")
        s = lax.axis_index("s")
        wid = s * NC + c
        z16 = jnp.zeros((16,), jnp.float32)

        def zrow(r, carry):
            for c0 in range(0, nout, 16):
                zbuf[r, pl.ds(c0, 16)] = z16
            return carry

        lax.fori_loop(0, rows_per, zrow, 0)
        pltpu.sync_copy(zbuf, acc.at[pl.ds(s * rows_per, rows_per)])
        gs = g_hbm.shape[0] // NS
        pltpu.sync_copy(g_hbm.at[pl.ds(s * gs, gs)],
                        gsh.at[pl.ds(s * gs, gs)])
        pltpu.sync_copy(src_hbm.at[wid], src_v)
        pltpu.sync_copy(dst_hbm.at[wid], dst_v)
        plsc.subcore_barrier()

        nhalf = n_chunks // KG

        def fire_gathers(hg, p, sem):
            for b in range(KG):
                pltpu.async_copy(
                    gsh.at[src_v.at[hg * KG + b]], gbuf.at[p, b], sem)

        def fire_scatters(hg, p, sem):
            for b in range(KG):
                pltpu.async_copy(
                    gbuf.at[p, b], acc.at[dst_v.at[hg * KG + b]], sem,
                    add=True)

        def drain(p, sem):
            for b in range(KG):
                pltpu.make_async_copy(
                    g_hbm.at[pl.ds(0, CH)], gbuf.at[p, b], sem).wait()

        fire_gathers(0, 0, gsem_a)

        def body(t, carry):
            hg = 2 * t
            fire_gathers(hg + 1, 1, gsem_b)
            drain(0, gsem_a)
            fire_scatters(hg, 0, ssem_a)
            drain(0, ssem_a)

            @pl.when(hg + 2 < nhalf)
            def _():
                fire_gathers(hg + 2, 0, gsem_a)

            drain(1, gsem_b)
            fire_scatters(hg + 1, 1, ssem_b)
            drain(1, ssem_b)
            return carry

        lax.fori_loop(0, nhalf // 2, body, 0)
        plsc.subcore_barrier()
        pltpu.sync_copy(acc.at[pl.ds(s * rows_per, rows_per)],
                        out_hbm.at[c, pl.ds(s * rows_per, rows_per)])

    return scatter


def kernel(x, edge_index, batch, params):
    n, d = x.shape
    e = edge_index.shape[1]
    nout = params['conv0_W'].shape[1]
    ng = 64
    bn = 2000

    n_chunks = -(-e // (NW * CH * NBUF)) * NBUF
    e_pad = NW * n_chunks * CH
    n_pad = -(-(n + 1) // (NS * 8)) * (NS * 8)

    batch2 = batch.reshape(n, 1)

    scatter = _make_scatter(n_pad, n_chunks, nout)

    out = None
    a = c = agg = None
    for i in range(5):
        if i == 0:
            g, a, c, srcr5, dstr5 = _dense(
                x, edge_index, e_pad, params['conv0_W'],
                params['fc_a0_W'], params['fc_a0_b'],
                params['fc_b0_W'], params['fc_b0_b'],
                params['fc_c0_W'], params['fc_c0_b'], bn)
            srcr = srcr5.reshape(NW, n_chunks, CH)
            dstr = dstr5.reshape(NW, n_chunks, CH)
        else:
            g, a, c = _fused_dense(
                a, agg, c, params[f'conv{i - 1}_b'],
                params[f'conv{i}_W'],
                params[f'fc_a{i}_W'], params[f'fc_a{i}_b'],
                params[f'fc_b{i}_W'], params[f'fc_b{i}_b'],
                params[f'fc_c{i}_W'], params[f'fc_c{i}_b'], bn)
        agg = scatter(g, srcr, dstr)
    out = _pool(a, agg, c, params['conv4_b'], batch2,
                params['fc1_W'], params['fc1_b'],
                params['fc2_W'], params['fc2_b'], ng, bn)
    return out


# speedup vs baseline: 1.0389x; 1.0389x over previous
import functools

import jax
import jax.numpy as jnp
from jax import lax
from jax.experimental import pallas as pl
from jax.experimental.pallas import tpu as pltpu
from jax.experimental.pallas import tpu_sc as plsc

NC = 2
NS = 16
NW = NC * NS
CH = 128


def _dense_body(e, n, echunk, h_ref, ei_ref, wg_ref, g_ref, src_ref,
                dst_ref):
    h = h_ref[...]
    g_ref[...] = jnp.dot(h, wg_ref[...], preferred_element_type=jnp.float32)
    i = pl.program_id(0)
    pos = i * echunk + jax.lax.broadcasted_iota(jnp.int32, (1, echunk), 1)
    valid = pos < e
    eb = ei_ref[...]
    src_ref[...] = jnp.where(valid, eb[0:1, :], 0).reshape(1, 1, echunk)
    dst_ref[...] = jnp.where(valid, eb[1:2, :], n).reshape(1, 1, echunk)


def _dense(h, ei, e_pad, wg, bn):
    n, fan = h.shape
    e = ei.shape[1]
    nout = wg.shape[1]
    grid = n // bn
    echunk = e_pad // grid
    full = lambda i: (0, 0)
    row = lambda i: (i, 0)
    return pl.pallas_call(
        functools.partial(_dense_body, e, n, echunk),
        grid=(grid,),
        in_specs=[
            pl.BlockSpec((bn, fan), row),
            pl.BlockSpec((2, echunk), lambda i: (0, i)),
            pl.BlockSpec((fan, nout), full),
        ],
        out_specs=[
            pl.BlockSpec((bn, nout), row),
            pl.BlockSpec((1, 1, echunk), lambda i: (i, 0, 0)),
            pl.BlockSpec((1, 1, echunk), lambda i: (i, 0, 0)),
        ],
        out_shape=[jax.ShapeDtypeStruct((n, nout), jnp.float32)]
        + [jax.ShapeDtypeStruct((grid, 1, echunk), jnp.int32)] * 2,
    )(h, ei, wg)


def _k1_body(a_ref, agg_ref, c_ref, pbias_ref, wg_ref, g_ref, h_ref):
    agg = agg_ref[...]
    b_ = jnp.maximum(agg[0] + agg[1] + pbias_ref[...], 0.0)
    h = jnp.concatenate([a_ref[...], b_, c_ref[...]], axis=1)
    h_ref[...] = h
    g_ref[...] = jnp.dot(h, wg_ref[...], preferred_element_type=jnp.float32)


def _k1(a, agg, c, pbias, wg, bn):
    n, nout = a.shape
    fan = 3 * nout
    grid = n // bn
    full = lambda i: (0, 0)
    row = lambda i: (i, 0)
    return pl.pallas_call(
        _k1_body,
        grid=(grid,),
        in_specs=[
            pl.BlockSpec((bn, nout), row),
            pl.BlockSpec((2, bn, nout), lambda i: (0, i, 0)),
            pl.BlockSpec((bn, nout), row),
            pl.BlockSpec((1, nout), full),
            pl.BlockSpec((fan, nout), full),
        ],
        out_specs=[
            pl.BlockSpec((bn, nout), row),
            pl.BlockSpec((bn, fan), row),
        ],
        out_shape=[jax.ShapeDtypeStruct((n, nout), jnp.float32),
                   jax.ShapeDtypeStruct((n, fan), jnp.float32)],
    )(a, agg, c, pbias.reshape(1, -1), wg)


def _k2_body(h_ref, wa_ref, ba_ref, wb_ref, bb_ref, wc_ref, bc_ref,
             a_ref, c_ref):
    h = h_ref[...]
    a = jnp.dot(h, wa_ref[...], preferred_element_type=jnp.float32) + ba_ref[...]
    a_ref[...] = jnp.maximum(a, 0.0)
    tb = jnp.dot(h, wb_ref[...], preferred_element_type=jnp.float32) + bb_ref[...]
    tc = jnp.dot(h, wc_ref[...], preferred_element_type=jnp.float32) + bc_ref[...]
    c_ref[...] = jnp.maximum(tb * tc, 0.0)


def _k2(h, wa, ba, wb, bb, wc, bc, nout, bn):
    n, fan = h.shape
    grid = n // bn
    full = lambda i: (0, 0)
    row = lambda i: (i, 0)
    return pl.pallas_call(
        _k2_body,
        grid=(grid,),
        in_specs=[
            pl.BlockSpec((bn, fan), row),
            pl.BlockSpec((fan, nout), full),
            pl.BlockSpec((1, nout), full),
            pl.BlockSpec((fan, nout), full),
            pl.BlockSpec((1, nout), full),
            pl.BlockSpec((fan, nout), full),
            pl.BlockSpec((1, nout), full),
        ],
        out_specs=[
            pl.BlockSpec((bn, nout), row),
            pl.BlockSpec((bn, nout), row),
        ],
        out_shape=[jax.ShapeDtypeStruct((n, nout), jnp.float32)] * 2,
    )(h, wa, ba.reshape(1, -1), wb, bb.reshape(1, -1), wc,
      bc.reshape(1, -1))


def _pool_body(a_ref, agg_ref, c_ref, bias_ref, batch_ref, w1_ref, b1_ref,
               w2_ref, b2_ref, out_ref, acc_ref):
    i = pl.program_id(0)
    agg = agg_ref[...]
    b_ = jnp.maximum(agg[0] + agg[1] + bias_ref[...], 0.0)
    h = jnp.concatenate([a_ref[...], b_, c_ref[...]], axis=1)
    gid = batch_ref[...]
    ng = acc_ref.shape[0]
    onehot = (gid == lax.broadcasted_iota(jnp.int32, (1, ng), 1)
              ).astype(jnp.float32)
    part = lax.dot_general(onehot, h, (((0,), (0,)), ((), ())),
                           preferred_element_type=jnp.float32)

    @pl.when(i == 0)
    def _():
        acc_ref[...] = jnp.zeros_like(acc_ref)

    acc_ref[...] += part

    @pl.when(i == pl.num_programs(0) - 1)
    def _():
        o = jnp.dot(acc_ref[...], w1_ref[...],
                    preferred_element_type=jnp.float32) + b1_ref[...]
        o = jnp.dot(o, w2_ref[...],
                    preferred_element_type=jnp.float32) + b2_ref[...]
        out_ref[...] = o


def _pool(a, agg, c, bias, batch2, w1, b1, w2, b2, ng, bn):
    n, nout = a.shape
    grid = n // bn
    nin = 3 * nout
    nh = w1.shape[1]
    return pl.pallas_call(
        _pool_body,
        grid=(grid,),
        in_specs=[
            pl.BlockSpec((bn, nout), lambda i: (i, 0)),
            pl.BlockSpec((2, bn, nout), lambda i: (0, i, 0)),
            pl.BlockSpec((bn, nout), lambda i: (i, 0)),
            pl.BlockSpec((1, nout), lambda i: (0, 0)),
            pl.BlockSpec((bn, 1), lambda i: (i, 0)),
            pl.BlockSpec((nin, nh), lambda i: (0, 0)),
            pl.BlockSpec((1, nh), lambda i: (0, 0)),
            pl.BlockSpec((nh, 1), lambda i: (0, 0)),
            pl.BlockSpec((1, 1), lambda i: (0, 0)),
        ],
        out_specs=pl.BlockSpec((ng, 1), lambda i: (0, 0)),
        out_shape=jax.ShapeDtypeStruct((ng, 1), jnp.float32),
        scratch_shapes=[pltpu.VMEM((ng, nin), jnp.float32)],
    )(a, agg, c, bias.reshape(1, -1), batch2, w1, b1.reshape(1, -1), w2,
      b2.reshape(1, -1))


NBUF = 8
KG = 4


def _make_scatter(n_pad, n_chunks, nout):
    rows_per = n_pad // NS
    ngroups = n_chunks // NBUF
    mesh = plsc.VectorSubcoreMesh(core_axis_name="c", subcore_axis_name="s")

    @functools.partial(
        pl.kernel, mesh=mesh,
        compiler_params=pltpu.CompilerParams(use_tc_tiling_on_sc=False),
        out_type=jax.ShapeDtypeStruct((NC, n_pad, nout), jnp.float32),
        scratch_types=[
            pltpu.VMEM((n_chunks, CH), jnp.int32),
            pltpu.VMEM((n_chunks, CH), jnp.int32),
            pltpu.VMEM((2, KG, CH, nout), jnp.float32),
            pltpu.VMEM_SHARED((n_pad, nout), jnp.float32),
            pltpu.VMEM_SHARED((n_pad, nout), jnp.float32),
            pltpu.SemaphoreType.DMA,
            pltpu.SemaphoreType.DMA,
            pltpu.SemaphoreType.DMA,
            pltpu.SemaphoreType.DMA,
        ],
    )
    def scatter(g_hbm, src_hbm, dst_hbm, zeros_hbm, out_hbm,
                src_v, dst_v, gbuf, acc, gsh, gsem_a, gsem_b, ssem_a,
                ssem_b):
        c = lax.axis_index("c")
        s = lax.axis_index("s")
        wid = s * NC + c
        pltpu.sync_copy(zeros_hbm, acc.at[pl.ds(s * rows_per, rows_per)])
        gs = g_hbm.shape[0] // NS
        pltpu.sync_copy(g_hbm.at[pl.ds(s * gs, gs)],
                        gsh.at[pl.ds(s * gs, gs)])
        pltpu.sync_copy(src_hbm.at[wid], src_v)
        pltpu.sync_copy(dst_hbm.at[wid], dst_v)
        plsc.subcore_barrier()

        nhalf = n_chunks // KG

        def fire_gathers(hg, p, sem):
            for b in range(KG):
                pltpu.async_copy(
                    gsh.at[src_v.at[hg * KG + b]], gbuf.at[p, b], sem)

        def fire_scatters(hg, p, sem):
            for b in range(KG):
                pltpu.async_copy(
                    gbuf.at[p, b], acc.at[dst_v.at[hg * KG + b]], sem,
                    add=True)

        def drain(p, sem):
            for b in range(KG):
                pltpu.make_async_copy(
                    g_hbm.at[pl.ds(0, CH)], gbuf.at[p, b], sem).wait()

        fire_gathers(0, 0, gsem_a)

        def body(t, carry):
            hg = 2 * t
            fire_gathers(hg + 1, 1, gsem_b)
            drain(0, gsem_a)
            fire_scatters(hg, 0, ssem_a)
            drain(0, ssem_a)

            @pl.when(hg + 2 < nhalf)
            def _():
                fire_gathers(hg + 2, 0, gsem_a)

            drain(1, gsem_b)
            fire_scatters(hg + 1, 1, ssem_b)
            drain(1, ssem_b)
            return carry

        lax.fori_loop(0, nhalf // 2, body, 0)
        plsc.subcore_barrier()
        pltpu.sync_copy(acc.at[pl.ds(s * rows_per, rows_per)],
                        out_hbm.at[c, pl.ds(s * rows_per, rows_per)])

    return scatter


def kernel(x, edge_index, batch, params):
    n, d = x.shape
    e = edge_index.shape[1]
    nout = params['conv0_W'].shape[1]
    ng = 64
    bn = 2000

    n_chunks = -(-e // (NW * CH * NBUF)) * NBUF
    e_pad = NW * n_chunks * CH
    n_pad = -(-(n + 1) // (NS * 8)) * (NS * 8)

    batch2 = batch.reshape(n, 1)
    zeros = jnp.zeros((n_pad // NS, nout), jnp.float32)

    scatter = _make_scatter(n_pad, n_chunks, nout)

    out = None
    a = c = agg = None
    for i in range(5):
        if i == 0:
            g, srcr5, dstr5 = _dense(x, edge_index, e_pad,
                                     params['conv0_W'], bn)
            srcr = srcr5.reshape(NW, n_chunks, CH)
            dstr = dstr5.reshape(NW, n_chunks, CH)
            h = x
        else:
            g, h = _k1(a, agg, c, params[f'conv{i - 1}_b'],
                       params[f'conv{i}_W'], bn)
        agg = scatter(g, srcr, dstr, zeros)
        a, c = _k2(h, params[f'fc_a{i}_W'], params[f'fc_a{i}_b'],
                   params[f'fc_b{i}_W'], params[f'fc_b{i}_b'],
                   params[f'fc_c{i}_W'], params[f'fc_c{i}_b'], nout, bn)
    out = _pool(a, agg, c, params['conv4_b'], batch2,
                params['fc1_W'], params['fc1_b'],
                params['fc2_W'], params['fc2_b'], ng, bn)
    return out


# speedup vs baseline: 1.0506x; 1.0113x over previous
import functools

import jax
import jax.numpy as jnp
from jax import lax
from jax.experimental import pallas as pl
from jax.experimental.pallas import tpu as pltpu
from jax.experimental.pallas import tpu_sc as plsc

NC = 2
NS = 16
NW = NC * NS
CH = 128


def _dense_body(e, n, echunk, h_ref, ei_ref, wg_ref, wa_ref, ba_ref,
                wb_ref, bb_ref, wc_ref, bc_ref, g_ref, ac_ref, src_ref,
                dst_ref):
    h = h_ref[...]
    g_ref[...] = jnp.dot(h, wg_ref[...], preferred_element_type=jnp.float32)
    a = jnp.dot(h, wa_ref[...], preferred_element_type=jnp.float32) + ba_ref[...]
    tb = jnp.dot(h, wb_ref[...], preferred_element_type=jnp.float32) + bb_ref[...]
    tc = jnp.dot(h, wc_ref[...], preferred_element_type=jnp.float32) + bc_ref[...]
    ac_ref[...] = jnp.concatenate(
        [jnp.maximum(a, 0.0), jnp.maximum(tb * tc, 0.0)], axis=1)
    i = pl.program_id(0)
    pos = i * echunk + jax.lax.broadcasted_iota(jnp.int32, (1, echunk), 1)
    valid = pos < e
    eb = ei_ref[...]
    src_ref[...] = jnp.where(valid, eb[0:1, :], 0).reshape(1, 1, echunk)
    dst_ref[...] = jnp.where(valid, eb[1:2, :], n).reshape(1, 1, echunk)


def _dense(h, ei, e_pad, wg, wa, ba, wb, bb, wc, bc, bn):
    n, fan = h.shape
    e = ei.shape[1]
    nout = wg.shape[1]
    grid = n // bn
    echunk = e_pad // grid
    full = lambda i: (0, 0)
    row = lambda i: (i, 0)
    return pl.pallas_call(
        functools.partial(_dense_body, e, n, echunk),
        grid=(grid,),
        in_specs=[
            pl.BlockSpec((bn, fan), row),
            pl.BlockSpec((2, echunk), lambda i: (0, i)),
            pl.BlockSpec((fan, nout), full),
            pl.BlockSpec((fan, nout), full),
            pl.BlockSpec((1, nout), full),
            pl.BlockSpec((fan, nout), full),
            pl.BlockSpec((1, nout), full),
            pl.BlockSpec((fan, nout), full),
            pl.BlockSpec((1, nout), full),
        ],
        out_specs=[
            pl.BlockSpec((bn, nout), row),
            pl.BlockSpec((bn, 2 * nout), row),
            pl.BlockSpec((1, 1, echunk), lambda i: (i, 0, 0)),
            pl.BlockSpec((1, 1, echunk), lambda i: (i, 0, 0)),
        ],
        out_shape=[jax.ShapeDtypeStruct((n, nout), jnp.float32),
                   jax.ShapeDtypeStruct((n, 2 * nout), jnp.float32)]
        + [jax.ShapeDtypeStruct((grid, 1, echunk), jnp.int32)] * 2,
    )(h, ei, wg, wa, ba.reshape(1, -1), wb, bb.reshape(1, -1), wc,
      bc.reshape(1, -1))


def _fused_body(ac_ref, agg_ref, pbias_ref, wg_ref, wa_ref, ba_ref,
                wb_ref, bb_ref, wc_ref, bc_ref, g_ref, ac_out_ref):
    ac = ac_ref[...]
    nout = g_ref.shape[1]
    agg = agg_ref[...]
    b_ = jnp.maximum(agg[0] + agg[1] + pbias_ref[...], 0.0)
    h = jnp.concatenate([ac[:, :nout], b_, ac[:, nout:]], axis=1)
    g_ref[...] = jnp.dot(h, wg_ref[...], preferred_element_type=jnp.float32)
    a = jnp.dot(h, wa_ref[...], preferred_element_type=jnp.float32) + ba_ref[...]
    tb = jnp.dot(h, wb_ref[...], preferred_element_type=jnp.float32) + bb_ref[...]
    tc = jnp.dot(h, wc_ref[...], preferred_element_type=jnp.float32) + bc_ref[...]
    ac_out_ref[...] = jnp.concatenate(
        [jnp.maximum(a, 0.0), jnp.maximum(tb * tc, 0.0)], axis=1)


def _fused(ac, agg, pbias, wg, wa, ba, wb, bb, wc, bc, bn):
    n = ac.shape[0]
    nout = wg.shape[1]
    fan = 3 * nout
    grid = n // bn
    full = lambda i: (0, 0)
    row = lambda i: (i, 0)
    return pl.pallas_call(
        _fused_body,
        grid=(grid,),
        in_specs=[
            pl.BlockSpec((bn, 2 * nout), row),
            pl.BlockSpec((2, bn, nout), lambda i: (0, i, 0)),
            pl.BlockSpec((1, nout), full),
            pl.BlockSpec((fan, nout), full),
            pl.BlockSpec((fan, nout), full),
            pl.BlockSpec((1, nout), full),
            pl.BlockSpec((fan, nout), full),
            pl.BlockSpec((1, nout), full),
            pl.BlockSpec((fan, nout), full),
            pl.BlockSpec((1, nout), full),
        ],
        out_specs=[
            pl.BlockSpec((bn, nout), row),
            pl.BlockSpec((bn, 2 * nout), row),
        ],
        out_shape=[jax.ShapeDtypeStruct((n, nout), jnp.float32),
                   jax.ShapeDtypeStruct((n, 2 * nout), jnp.float32)],
    )(ac, agg, pbias.reshape(1, -1), wg, wa, ba.reshape(1, -1), wb,
      bb.reshape(1, -1), wc, bc.reshape(1, -1))


def _pool_body(ac_ref, agg_ref, bias_ref, batch_ref, w1_ref, b1_ref,
               w2_ref, b2_ref, out_ref, acc_ref):
    i = pl.program_id(0)
    ac = ac_ref[...]
    nout = agg_ref.shape[2]
    agg = agg_ref[...]
    b_ = jnp.maximum(agg[0] + agg[1] + bias_ref[...], 0.0)
    h = jnp.concatenate([ac[:, :nout], b_, ac[:, nout:]], axis=1)
    gid = batch_ref[...]
    ng = acc_ref.shape[0]
    onehot = (gid == lax.broadcasted_iota(jnp.int32, (1, ng), 1)
              ).astype(jnp.float32)
    part = lax.dot_general(onehot, h, (((0,), (0,)), ((), ())),
                           preferred_element_type=jnp.float32)

    @pl.when(i == 0)
    def _():
        acc_ref[...] = jnp.zeros_like(acc_ref)

    acc_ref[...] += part

    @pl.when(i == pl.num_programs(0) - 1)
    def _():
        o = jnp.dot(acc_ref[...], w1_ref[...],
                    preferred_element_type=jnp.float32) + b1_ref[...]
        o = jnp.dot(o, w2_ref[...],
                    preferred_element_type=jnp.float32) + b2_ref[...]
        out_ref[...] = o


def _pool(ac, agg, bias, batch2, w1, b1, w2, b2, ng, bn):
    n = ac.shape[0]
    nout = agg.shape[2]
    grid = n // bn
    nin = 3 * nout
    nh = w1.shape[1]
    return pl.pallas_call(
        _pool_body,
        grid=(grid,),
        in_specs=[
            pl.BlockSpec((bn, 2 * nout), lambda i: (i, 0)),
            pl.BlockSpec((2, bn, nout), lambda i: (0, i, 0)),
            pl.BlockSpec((1, nout), lambda i: (0, 0)),
            pl.BlockSpec((bn, 1), lambda i: (i, 0)),
            pl.BlockSpec((nin, nh), lambda i: (0, 0)),
            pl.BlockSpec((1, nh), lambda i: (0, 0)),
            pl.BlockSpec((nh, 1), lambda i: (0, 0)),
            pl.BlockSpec((1, 1), lambda i: (0, 0)),
        ],
        out_specs=pl.BlockSpec((ng, 1), lambda i: (0, 0)),
        out_shape=jax.ShapeDtypeStruct((ng, 1), jnp.float32),
        scratch_shapes=[pltpu.VMEM((ng, nin), jnp.float32)],
    )(ac, agg, bias.reshape(1, -1), batch2, w1, b1.reshape(1, -1), w2,
      b2.reshape(1, -1))


NBUF = 8
KG = 4


def _make_scatter(n_pad, n_chunks, nout):
    rows_per = n_pad // NS
    ngroups = n_chunks // NBUF
    mesh = plsc.VectorSubcoreMesh(core_axis_name="c", subcore_axis_name="s")

    @functools.partial(
        pl.kernel, mesh=mesh,
        compiler_params=pltpu.CompilerParams(use_tc_tiling_on_sc=False),
        out_type=jax.ShapeDtypeStruct((NC, n_pad, nout), jnp.float32),
        scratch_types=[
            pltpu.VMEM((n_chunks, CH), jnp.int32),
            pltpu.VMEM((n_chunks, CH), jnp.int32),
            pltpu.VMEM((2, KG, CH, nout), jnp.float32),
            pltpu.VMEM_SHARED((n_pad, nout), jnp.float32),
            pltpu.VMEM_SHARED((n_pad, nout), jnp.float32),
            pltpu.SemaphoreType.DMA,
            pltpu.SemaphoreType.DMA,
            pltpu.SemaphoreType.DMA,
            pltpu.SemaphoreType.DMA,
        ],
    )
    def scatter(g_hbm, src_hbm, dst_hbm, zeros_hbm, out_hbm,
                src_v, dst_v, gbuf, acc, gsh, gsem_a, gsem_b, ssem_a,
                ssem_b):
        c = lax.axis_index("c")
        s = lax.axis_index("s")
        wid = s * NC + c
        pltpu.sync_copy(zeros_hbm, acc.at[pl.ds(s * rows_per, rows_per)])
        gs = g_hbm.shape[0] // NS
        pltpu.sync_copy(g_hbm.at[pl.ds(s * gs, gs)],
                        gsh.at[pl.ds(s * gs, gs)])
        pltpu.sync_copy(src_hbm.at[wid], src_v)
        pltpu.sync_copy(dst_hbm.at[wid], dst_v)
        plsc.subcore_barrier()

        nhalf = n_chunks // KG

        def fire_gathers(hg, p, sem):
            for b in range(KG):
                pltpu.async_copy(
                    gsh.at[src_v.at[hg * KG + b]], gbuf.at[p, b], sem)

        def fire_scatters(hg, p, sem):
            for b in range(KG):
                pltpu.async_copy(
                    gbuf.at[p, b], acc.at[dst_v.at[hg * KG + b]], sem,
                    add=True)

        def drain(p, sem):
            for b in range(KG):
                pltpu.make_async_copy(
                    g_hbm.at[pl.ds(0, CH)], gbuf.at[p, b], sem).wait()

        fire_gathers(0, 0, gsem_a)

        def body(t, carry):
            hg = 2 * t
            fire_gathers(hg + 1, 1, gsem_b)
            drain(0, gsem_a)
            fire_scatters(hg, 0, ssem_a)
            drain(0, ssem_a)

            @pl.when(hg + 2 < nhalf)
            def _():
                fire_gathers(hg + 2, 0, gsem_a)

            drain(1, gsem_b)
            fire_scatters(hg + 1, 1, ssem_b)
            drain(1, ssem_b)
            return carry

        lax.fori_loop(0, nhalf // 2, body, 0)
        plsc.subcore_barrier()
        pltpu.sync_copy(acc.at[pl.ds(s * rows_per, rows_per)],
                        out_hbm.at[c, pl.ds(s * rows_per, rows_per)])

    return scatter


def kernel(x, edge_index, batch, params):
    n, d = x.shape
    e = edge_index.shape[1]
    nout = params['conv0_W'].shape[1]
    ng = 64
    bn = 2000

    n_chunks = -(-e // (NW * CH * NBUF)) * NBUF
    e_pad = NW * n_chunks * CH
    n_pad = -(-(n + 1) // (NS * 8)) * (NS * 8)

    batch2 = batch.reshape(n, 1)
    zeros = jnp.zeros((n_pad // NS, nout), jnp.float32)

    scatter = _make_scatter(n_pad, n_chunks, nout)

    out = None
    ac = agg = None
    for i in range(5):
        if i == 0:
            g, ac, srcr5, dstr5 = _dense(
                x, edge_index, e_pad, params['conv0_W'],
                params['fc_a0_W'], params['fc_a0_b'],
                params['fc_b0_W'], params['fc_b0_b'],
                params['fc_c0_W'], params['fc_c0_b'], bn)
            srcr = srcr5.reshape(NW, n_chunks, CH)
            dstr = dstr5.reshape(NW, n_chunks, CH)
        else:
            g, ac = _fused(ac, agg, params[f'conv{i - 1}_b'],
                           params[f'conv{i}_W'],
                           params[f'fc_a{i}_W'], params[f'fc_a{i}_b'],
                           params[f'fc_b{i}_W'], params[f'fc_b{i}_b'],
                           params[f'fc_c{i}_W'], params[f'fc_c{i}_b'], bn)
        agg = scatter(g, srcr, dstr, zeros)
    out = _pool(ac, agg, params['conv4_b'], batch2,
                params['fc1_W'], params['fc1_b'],
                params['fc2_W'], params['fc2_b'], ng, bn)
    return out


# speedup vs baseline: 1.0517x; 1.0011x over previous
import functools

import jax
import jax.numpy as jnp
from jax import lax
from jax.experimental import pallas as pl
from jax.experimental.pallas import tpu as pltpu
from jax.experimental.pallas import tpu_sc as plsc

NC = 2
NS = 16
NW = NC * NS
CH = 128


def _dense_body(e, n, echunk, h_ref, ei_ref, wg_ref, wa_ref, ba_ref,
                wb_ref, bb_ref, wc_ref, bc_ref, g_ref, ac_ref, src_ref,
                dst_ref):
    h = h_ref[...]
    g_ref[...] = jnp.dot(h, wg_ref[...], preferred_element_type=jnp.float32)
    a = jnp.dot(h, wa_ref[...], preferred_element_type=jnp.float32) + ba_ref[...]
    tb = jnp.dot(h, wb_ref[...], preferred_element_type=jnp.float32) + bb_ref[...]
    tc = jnp.dot(h, wc_ref[...], preferred_element_type=jnp.float32) + bc_ref[...]
    ac_ref[...] = jnp.concatenate(
        [jnp.maximum(a, 0.0), jnp.maximum(tb * tc, 0.0)], axis=1)
    i = pl.program_id(0)
    pos = i * echunk + jax.lax.broadcasted_iota(jnp.int32, (1, echunk), 1)
    valid = pos < e
    eb = ei_ref[...]
    src_ref[...] = jnp.where(valid, eb[0:1, :], 0).reshape(1, 1, echunk)
    dst_ref[...] = jnp.where(valid, eb[1:2, :], n).reshape(1, 1, echunk)


def _dense(h, ei, e_pad, wg, wa, ba, wb, bb, wc, bc, bn):
    n, fan = h.shape
    e = ei.shape[1]
    nout = wg.shape[1]
    grid = n // bn
    echunk = e_pad // grid
    full = lambda i: (0, 0)
    row = lambda i: (i, 0)
    return pl.pallas_call(
        functools.partial(_dense_body, e, n, echunk),
        grid=(grid,),
        in_specs=[
            pl.BlockSpec((bn, fan), row),
            pl.BlockSpec((2, echunk), lambda i: (0, i)),
            pl.BlockSpec((fan, nout), full),
            pl.BlockSpec((fan, nout), full),
            pl.BlockSpec((1, nout), full),
            pl.BlockSpec((fan, nout), full),
            pl.BlockSpec((1, nout), full),
            pl.BlockSpec((fan, nout), full),
            pl.BlockSpec((1, nout), full),
        ],
        out_specs=[
            pl.BlockSpec((bn, nout), row),
            pl.BlockSpec((bn, 2 * nout), row),
            pl.BlockSpec((1, 1, echunk), lambda i: (i, 0, 0)),
            pl.BlockSpec((1, 1, echunk), lambda i: (i, 0, 0)),
        ],
        out_shape=[jax.ShapeDtypeStruct((n, nout), jnp.float32),
                   jax.ShapeDtypeStruct((n, 2 * nout), jnp.float32)]
        + [jax.ShapeDtypeStruct((grid, 1, echunk), jnp.int32)] * 2,
    )(h, ei, wg, wa, ba.reshape(1, -1), wb, bb.reshape(1, -1), wc,
      bc.reshape(1, -1))


def _fused_body(ac_ref, agg_ref, pbias_ref, wg_ref, wa_ref, ba_ref,
                wb_ref, bb_ref, wc_ref, bc_ref, g_ref, ac_out_ref):
    ac = ac_ref[...]
    nout = g_ref.shape[1]
    agg = agg_ref[...]
    b_ = jnp.maximum(agg[0] + agg[1] + pbias_ref[...], 0.0)
    h = jnp.concatenate([ac[:, :nout], b_, ac[:, nout:]], axis=1)
    g_ref[...] = jnp.dot(h, wg_ref[...], preferred_element_type=jnp.float32)
    a = jnp.dot(h, wa_ref[...], preferred_element_type=jnp.float32) + ba_ref[...]
    tb = jnp.dot(h, wb_ref[...], preferred_element_type=jnp.float32) + bb_ref[...]
    tc = jnp.dot(h, wc_ref[...], preferred_element_type=jnp.float32) + bc_ref[...]
    ac_out_ref[...] = jnp.concatenate(
        [jnp.maximum(a, 0.0), jnp.maximum(tb * tc, 0.0)], axis=1)


def _fused(ac, agg, pbias, wg, wa, ba, wb, bb, wc, bc, bn):
    n = ac.shape[0]
    nout = wg.shape[1]
    fan = 3 * nout
    grid = n // bn
    full = lambda i: (0, 0)
    row = lambda i: (i, 0)
    return pl.pallas_call(
        _fused_body,
        grid=(grid,),
        in_specs=[
            pl.BlockSpec((bn, 2 * nout), row),
            pl.BlockSpec((2, bn, nout), lambda i: (0, i, 0)),
            pl.BlockSpec((1, nout), full),
            pl.BlockSpec((fan, nout), full),
            pl.BlockSpec((fan, nout), full),
            pl.BlockSpec((1, nout), full),
            pl.BlockSpec((fan, nout), full),
            pl.BlockSpec((1, nout), full),
            pl.BlockSpec((fan, nout), full),
            pl.BlockSpec((1, nout), full),
        ],
        out_specs=[
            pl.BlockSpec((bn, nout), row),
            pl.BlockSpec((bn, 2 * nout), row),
        ],
        out_shape=[jax.ShapeDtypeStruct((n, nout), jnp.float32),
                   jax.ShapeDtypeStruct((n, 2 * nout), jnp.float32)],
    )(ac, agg, pbias.reshape(1, -1), wg, wa, ba.reshape(1, -1), wb,
      bb.reshape(1, -1), wc, bc.reshape(1, -1))


def _pool_body(ac_ref, agg_ref, bias_ref, batch_ref, w1_ref, b1_ref,
               w2_ref, b2_ref, out_ref, acc_ref):
    i = pl.program_id(0)
    ac = ac_ref[...]
    nout = agg_ref.shape[2]
    agg = agg_ref[...]
    b_ = jnp.maximum(agg[0] + agg[1] + bias_ref[...], 0.0)
    h = jnp.concatenate([ac[:, :nout], b_, ac[:, nout:]], axis=1)
    gid = batch_ref[...]
    ng = acc_ref.shape[0]
    onehot = (gid == lax.broadcasted_iota(jnp.int32, (1, ng), 1)
              ).astype(jnp.float32)
    part = lax.dot_general(onehot, h, (((0,), (0,)), ((), ())),
                           preferred_element_type=jnp.float32)

    @pl.when(i == 0)
    def _():
        acc_ref[...] = jnp.zeros_like(acc_ref)

    acc_ref[...] += part

    @pl.when(i == pl.num_programs(0) - 1)
    def _():
        o = jnp.dot(acc_ref[...], w1_ref[...],
                    preferred_element_type=jnp.float32) + b1_ref[...]
        o = jnp.dot(o, w2_ref[...],
                    preferred_element_type=jnp.float32) + b2_ref[...]
        out_ref[...] = o


def _pool(ac, agg, bias, batch2, w1, b1, w2, b2, ng, bn):
    n = ac.shape[0]
    nout = agg.shape[2]
    grid = n // bn
    nin = 3 * nout
    nh = w1.shape[1]
    return pl.pallas_call(
        _pool_body,
        grid=(grid,),
        in_specs=[
            pl.BlockSpec((bn, 2 * nout), lambda i: (i, 0)),
            pl.BlockSpec((2, bn, nout), lambda i: (0, i, 0)),
            pl.BlockSpec((1, nout), lambda i: (0, 0)),
            pl.BlockSpec((bn, 1), lambda i: (i, 0)),
            pl.BlockSpec((nin, nh), lambda i: (0, 0)),
            pl.BlockSpec((1, nh), lambda i: (0, 0)),
            pl.BlockSpec((nh, 1), lambda i: (0, 0)),
            pl.BlockSpec((1, 1), lambda i: (0, 0)),
        ],
        out_specs=pl.BlockSpec((ng, 1), lambda i: (0, 0)),
        out_shape=jax.ShapeDtypeStruct((ng, 1), jnp.float32),
        scratch_shapes=[pltpu.VMEM((ng, nin), jnp.float32)],
    )(ac, agg, bias.reshape(1, -1), batch2, w1, b1.reshape(1, -1), w2,
      b2.reshape(1, -1))


NBUF = 8
KG = 8


def _make_scatter(n_pad, n_chunks, nout):
    rows_per = n_pad // NS
    ngroups = n_chunks // NBUF
    mesh = plsc.VectorSubcoreMesh(core_axis_name="c", subcore_axis_name="s")

    @functools.partial(
        pl.kernel, mesh=mesh,
        compiler_params=pltpu.CompilerParams(use_tc_tiling_on_sc=False),
        out_type=jax.ShapeDtypeStruct((NC, n_pad, nout), jnp.float32),
        scratch_types=[
            pltpu.VMEM((n_chunks, CH), jnp.int32),
            pltpu.VMEM((n_chunks, CH), jnp.int32),
            pltpu.VMEM((2, KG, CH, nout), jnp.float32),
            pltpu.VMEM_SHARED((n_pad, nout), jnp.float32),
            pltpu.VMEM_SHARED((n_pad, nout), jnp.float32),
            pltpu.SemaphoreType.DMA,
            pltpu.SemaphoreType.DMA,
            pltpu.SemaphoreType.DMA,
            pltpu.SemaphoreType.DMA,
        ],
    )
    def scatter(g_hbm, src_hbm, dst_hbm, zeros_hbm, out_hbm,
                src_v, dst_v, gbuf, acc, gsh, gsem_a, gsem_b, ssem_a,
                ssem_b):
        c = lax.axis_index("c")
        s = lax.axis_index("s")
        wid = s * NC + c
        pltpu.sync_copy(zeros_hbm, acc.at[pl.ds(s * rows_per, rows_per)])
        gs = g_hbm.shape[0] // NS
        pltpu.sync_copy(g_hbm.at[pl.ds(s * gs, gs)],
                        gsh.at[pl.ds(s * gs, gs)])
        pltpu.sync_copy(src_hbm.at[wid], src_v)
        pltpu.sync_copy(dst_hbm.at[wid], dst_v)
        plsc.subcore_barrier()

        nhalf = n_chunks // KG

        def fire_gathers(hg, p, sem):
            for b in range(KG):
                pltpu.async_copy(
                    gsh.at[src_v.at[hg * KG + b]], gbuf.at[p, b], sem)

        def fire_scatters(hg, p, sem):
            for b in range(KG):
                pltpu.async_copy(
                    gbuf.at[p, b], acc.at[dst_v.at[hg * KG + b]], sem,
                    add=True)

        def drain(p, sem):
            for b in range(KG):
                pltpu.make_async_copy(
                    g_hbm.at[pl.ds(0, CH)], gbuf.at[p, b], sem).wait()

        fire_gathers(0, 0, gsem_a)

        def body(t, carry):
            hg = 2 * t
            fire_gathers(hg + 1, 1, gsem_b)
            drain(0, gsem_a)
            fire_scatters(hg, 0, ssem_a)
            drain(0, ssem_a)

            @pl.when(hg + 2 < nhalf)
            def _():
                fire_gathers(hg + 2, 0, gsem_a)

            drain(1, gsem_b)
            fire_scatters(hg + 1, 1, ssem_b)
            drain(1, ssem_b)
            return carry

        lax.fori_loop(0, nhalf // 2, body, 0)
        plsc.subcore_barrier()
        pltpu.sync_copy(acc.at[pl.ds(s * rows_per, rows_per)],
                        out_hbm.at[c, pl.ds(s * rows_per, rows_per)])

    return scatter


def kernel(x, edge_index, batch, params):
    n, d = x.shape
    e = edge_index.shape[1]
    nout = params['conv0_W'].shape[1]
    ng = 64
    bn = 2000

    n_chunks = -(-e // (NW * CH * NBUF)) * NBUF
    e_pad = NW * n_chunks * CH
    n_pad = -(-(n + 1) // (NS * 8)) * (NS * 8)

    batch2 = batch.reshape(n, 1)
    zeros = jnp.zeros((n_pad // NS, nout), jnp.float32)

    scatter = _make_scatter(n_pad, n_chunks, nout)

    out = None
    ac = agg = None
    for i in range(5):
        if i == 0:
            g, ac, srcr5, dstr5 = _dense(
                x, edge_index, e_pad, params['conv0_W'],
                params['fc_a0_W'], params['fc_a0_b'],
                params['fc_b0_W'], params['fc_b0_b'],
                params['fc_c0_W'], params['fc_c0_b'], bn)
            srcr = srcr5.reshape(NW, n_chunks, CH)
            dstr = dstr5.reshape(NW, n_chunks, CH)
        else:
            g, ac = _fused(ac, agg, params[f'conv{i - 1}_b'],
                           params[f'conv{i}_W'],
                           params[f'fc_a{i}_W'], params[f'fc_a{i}_b'],
                           params[f'fc_b{i}_W'], params[f'fc_b{i}_b'],
                           params[f'fc_c{i}_W'], params[f'fc_c{i}_b'], bn)
        agg = scatter(g, srcr, dstr, zeros)
    out = _pool(ac, agg, params['conv4_b'], batch2,
                params['fc1_W'], params['fc1_b'],
                params['fc2_W'], params['fc2_b'], ng, bn)
    return out
